# baseline (speedup 1.0000x reference)
import functools

import jax
import jax.numpy as jnp
from jax import lax
from jax.experimental import pallas as pl
from jax.experimental.pallas import tpu as pltpu
from jax.experimental.pallas import tpu_sc as plsc

_B = 16384
_D = 512
_C = 1000

_NC = 2
_NS = 16
_NW = _NC * _NS
_BPW = _B // _NW
_CH = 32
_NCHUNK = _BPW // _CH
_NBUF = 3


def _body(feats_hbm, labels_hbm, centers_hbm, out_hbm,
          idx_all, fb0, fb1, fb2, gb0, gb1, gb2, acc_v,
          sf0, sf1, sf2, sg0, sg1, sg2, sl):
    cid = lax.axis_index("c")
    sid = lax.axis_index("s")
    wid = sid * _NC + cid
    base = wid * _BPW

    lcps = [
        pltpu.async_copy(labels_hbm.at[pl.ds(base + k * _CH, _CH)],
                         idx_all.at[k], sl)
        for k in range(_NCHUNK)
    ]
    for cp in lcps:
        cp.wait()

    fbs = (fb0, fb1, fb2)
    gbs = (gb0, gb1, gb2)
    sfs = (sf0, sf1, sf2)
    sgs = (sg0, sg1, sg2)

    def start_feats(k):
        return pltpu.async_copy(
            feats_hbm.at[pl.ds(base + k * _CH, _CH)], fbs[k % _NBUF],
            sfs[k % _NBUF])

    def start_gather(k):
        return pltpu.async_copy(
            centers_hbm.at[idx_all.at[k]], gbs[k % _NBUF], sgs[k % _NBUF])

    def compute_chunk(fbuf, gbuf, acc4):
        def row_body(r, acc4):
            a0, a1, a2, a3 = acc4
            for j in range(_D // 64):
                x0 = fbuf[r, pl.ds((4 * j + 0) * 16, 16)] - \
                    gbuf[r, pl.ds((4 * j + 0) * 16, 16)]
                a0 = a0 + x0 * x0
                x1 = fbuf[r, pl.ds((4 * j + 1) * 16, 16)] - \
                    gbuf[r, pl.ds((4 * j + 1) * 16, 16)]
                a1 = a1 + x1 * x1
                x2 = fbuf[r, pl.ds((4 * j + 2) * 16, 16)] - \
                    gbuf[r, pl.ds((4 * j + 2) * 16, 16)]
                a2 = a2 + x2 * x2
                x3 = fbuf[r, pl.ds((4 * j + 3) * 16, 16)] - \
                    gbuf[r, pl.ds((4 * j + 3) * 16, 16)]
                a3 = a3 + x3 * x3
            return (a0, a1, a2, a3)
        return plsc.parallel_loop(0, _CH, carry=acc4)(row_body)

    cpf = {}
    cpg = {}
    for k in range(_NBUF):
        cpf[k] = start_feats(k)
        cpg[k] = start_gather(k)

    z = jnp.zeros((16,), jnp.float32)
    acc4 = (z, z, z, z)
    for k in range(_NCHUNK):
        b = k % _NBUF
        cpf[k].wait()
        cpg[k].wait()
        acc4 = compute_chunk(fbs[b], gbs[b], acc4)
        if k + _NBUF < _NCHUNK:
            cpf[k + _NBUF] = start_feats(k + _NBUF)
            cpg[k + _NBUF] = start_gather(k + _NBUF)

    acc_v[...] = acc4[0] + acc4[1] + acc4[2] + acc4[3]
    pltpu.sync_copy(acc_v, out_hbm.at[wid])


_mesh = plsc.VectorSubcoreMesh(core_axis_name="c", subcore_axis_name="s")

_sc_partials = functools.partial(
    pl.kernel,
    out_type=jax.ShapeDtypeStruct((_NW, 16), jnp.float32),
    mesh=_mesh,
    scratch_types=[
        pltpu.VMEM((_NCHUNK, _CH), jnp.int32),
        pltpu.VMEM((_CH, _D), jnp.float32),
        pltpu.VMEM((_CH, _D), jnp.float32),
        pltpu.VMEM((_CH, _D), jnp.float32),
        pltpu.VMEM((_CH, _D), jnp.float32),
        pltpu.VMEM((_CH, _D), jnp.float32),
        pltpu.VMEM((_CH, _D), jnp.float32),
        pltpu.VMEM((16,), jnp.float32),
        pltpu.SemaphoreType.DMA,
        pltpu.SemaphoreType.DMA,
        pltpu.SemaphoreType.DMA,
        pltpu.SemaphoreType.DMA,
        pltpu.SemaphoreType.DMA,
        pltpu.SemaphoreType.DMA,
        pltpu.SemaphoreType.DMA,
    ],
)(_body)


@jax.jit
def kernel(feats, labels, centers):
    partials = _sc_partials(feats, labels.astype(jnp.int32), centers)
    return jnp.sum(partials) / jnp.float32(_B * _D)


# speedup vs baseline: 1.1220x; 1.1220x over previous
import functools

import jax
import jax.numpy as jnp
from jax import lax
from jax.experimental import pallas as pl
from jax.experimental.pallas import tpu as pltpu
from jax.experimental.pallas import tpu_sc as plsc

_B = 16384
_D = 512
_C = 1000

_NC = 2
_NS = 16
_NW = _NC * _NS
_BPW = _B // _NW
_CH = 32
_NCHUNK = _BPW // _CH
_NBUF = 2


def _body(feats_hbm, labels_hbm, centers_hbm, out_hbm,
          idx_all, fb0, fb1, gb0, gb1, acc_v, tab_sh,
          sf0, sf1, sg0, sg1, sl):
    cid = lax.axis_index("c")
    sid = lax.axis_index("s")
    wid = sid * _NC + cid
    base = wid * _BPW

    fbs = (fb0, fb1)
    gbs = (gb0, gb1)
    sfs = (sf0, sf1)
    sgs = (sg0, sg1)

    @pl.when(sid < 8)
    def _():
        o = sid * (_C // 8) * _D
        pltpu.sync_copy(centers_hbm.at[pl.ds(o, (_C // 8) * _D)],
                        tab_sh.at[pl.ds(o, (_C // 8) * _D)])

    lcps = [
        pltpu.async_copy(labels_hbm.at[pl.ds(base + k * _CH, _CH)],
                         idx_all.at[k], sl)
        for k in range(_NCHUNK)
    ]
    for cp in lcps:
        cp.wait()
    plsc.subcore_barrier()

    def issue(k, b):
        pltpu.async_copy(feats_hbm.at[pl.ds(base + k * _CH, _CH)],
                         fbs[b], sfs[b])
        for j in range(_CH // 16):
            ixv = idx_all[k, pl.ds(j * 16, 16)]
            for l in range(16):
                off = pl.multiple_of(ixv[l] * _D, _D)
                pltpu.async_copy(tab_sh.at[pl.ds(off, _D)],
                                 gbs[b].at[pl.ds((j * 16 + l) * _D, _D)],
                                 sgs[b])

    def drain(b):
        pltpu.make_async_copy(feats_hbm.at[pl.ds(0, _CH)],
                              fbs[b], sfs[b]).wait()
        pltpu.make_async_copy(centers_hbm.at[pl.ds(0, _CH * _D)],
                              gbs[b], sgs[b]).wait()

    def compute_chunk(fbuf, gbuf, acc4):
        def row_body(r, acc4):
            a0, a1, a2, a3 = acc4
            g0 = r * _D
            for j in range(_D // 64):
                x0 = fbuf[r, pl.ds((4 * j + 0) * 16, 16)] - \
                    gbuf[pl.ds(g0 + (4 * j + 0) * 16, 16)]
                a0 = a0 + x0 * x0
                x1 = fbuf[r, pl.ds((4 * j + 1) * 16, 16)] - \
                    gbuf[pl.ds(g0 + (4 * j + 1) * 16, 16)]
                a1 = a1 + x1 * x1
                x2 = fbuf[r, pl.ds((4 * j + 2) * 16, 16)] - \
                    gbuf[pl.ds(g0 + (4 * j + 2) * 16, 16)]
                a2 = a2 + x2 * x2
                x3 = fbuf[r, pl.ds((4 * j + 3) * 16, 16)] - \
                    gbuf[pl.ds(g0 + (4 * j + 3) * 16, 16)]
                a3 = a3 + x3 * x3
            return (a0, a1, a2, a3)
        return plsc.parallel_loop(0, _CH, carry=acc4)(row_body)

    for b in range(_NBUF):
        issue(b, b)

    z = jnp.zeros((16,), jnp.float32)

    def group_body(gi, acc4):
        k0 = gi * _NBUF
        for b in range(_NBUF):
            k = k0 + b
            drain(b)
            acc4 = compute_chunk(fbs[b], gbs[b], acc4)

            @pl.when(k + _NBUF < _NCHUNK)
            def _():
                issue(k + _NBUF, b)
        return acc4

    acc4 = lax.fori_loop(0, _NCHUNK // _NBUF, group_body, (z, z, z, z))

    acc_v[...] = acc4[0] + acc4[1] + acc4[2] + acc4[3]
    pltpu.sync_copy(acc_v, out_hbm.at[wid])


_mesh = plsc.VectorSubcoreMesh(core_axis_name="c", subcore_axis_name="s")

_sc_partials = functools.partial(
    pl.kernel,
    out_type=jax.ShapeDtypeStruct((_NW, 16), jnp.float32),
    mesh=_mesh,
    scratch_types=[
        pltpu.VMEM((_NCHUNK, _CH), jnp.int32),
        pltpu.VMEM((_CH, _D), jnp.float32),
        pltpu.VMEM((_CH, _D), jnp.float32),
        pltpu.VMEM((_CH * _D,), jnp.float32),
        pltpu.VMEM((_CH * _D,), jnp.float32),
        pltpu.VMEM((16,), jnp.float32),
        pltpu.VMEM_SHARED((_C * _D,), jnp.float32),
        pltpu.SemaphoreType.DMA,
        pltpu.SemaphoreType.DMA,
        pltpu.SemaphoreType.DMA,
        pltpu.SemaphoreType.DMA,
        pltpu.SemaphoreType.DMA,
    ],
)(_body)


@jax.jit
def kernel(feats, labels, centers):
    partials = _sc_partials(feats, labels.astype(jnp.int32),
                            centers.reshape(_C * _D))
    return jnp.sum(partials) / jnp.float32(_B * _D)
